# Initial kernel scaffold; baseline (speedup 1.0000x reference)
import jax
import jax.numpy as jnp
from jax.experimental import pallas as pl


def kernel(g0_adj, g1_adj, h0, h1_feat, data, dateset_index, edge, params):
    raise NotImplementedError("write your pallas kernel here")


# speedup vs baseline: 3.7082x; 3.7082x over previous
import functools

import jax
import jax.numpy as jnp
from jax import lax
from jax.experimental import pallas as pl

N = 4096
P = 4096
DS = 2048
H = 128
D2 = 256
KNN = 10
BLK = 256
GRID = N // BLK
NEG = -3.0e38


def _elu(x):
    return jnp.where(x > 0, x, jnp.exp(jnp.minimum(x, 0.0)) - 1.0)


def _dot_t(a, b, prec=None):
    return lax.dot_general(a, b, (((1,), (1,)), ((), ())),
                           preferred_element_type=jnp.float32,
                           precision=prec)


def _dot(a, b, prec=None):
    return lax.dot_general(a, b, (((1,), (0,)), ((), ())),
                           preferred_element_type=jnp.float32,
                           precision=prec)


def _transform_body(h_ref, wt_ref, w0_ref, w1_ref, s0_ref, s1_ref):
    ht = _dot_t(h_ref[...], wt_ref[...])
    s0_ref[...] = _dot(ht, w0_ref[...])
    s1_ref[...] = _dot(ht, w1_ref[...])


def _transform(h, wt, w0, w1):
    din = h.shape[1]
    return pl.pallas_call(
        _transform_body,
        grid=(GRID,),
        in_specs=[
            pl.BlockSpec((BLK, din), lambda i: (i, 0)),
            pl.BlockSpec((H, din), lambda i: (0, 0)),
            pl.BlockSpec((H, H), lambda i: (0, 0)),
            pl.BlockSpec((H, H), lambda i: (0, 0)),
        ],
        out_specs=[
            pl.BlockSpec((BLK, H), lambda i: (i, 0)),
            pl.BlockSpec((BLK, H), lambda i: (i, 0)),
        ],
        out_shape=[
            jax.ShapeDtypeStruct((N, H), jnp.float32),
            jax.ShapeDtypeStruct((N, H), jnp.float32),
        ],
    )(h, wt, w0, w1)


def _han_body(a0_ref, a1_ref, s0_ref, s1_ref, b0_ref, b1_ref,
              aw1_ref, ab1_ref, aw2_ref, e0_ref, e1_ref, t_ref):
    i = pl.program_id(0)
    e0 = _elu(_dot(a0_ref[...], s0_ref[...]) + b0_ref[...])
    e1 = _elu(_dot(a1_ref[...], s1_ref[...]) + b1_ref[...])
    e0_ref[...] = e0
    e1_ref[...] = e1
    aw1 = aw1_ref[...]
    ab1 = ab1_ref[...]
    aw2 = aw2_ref[...]
    t0 = jnp.sum(_dot_t(jnp.tanh(_dot_t(e0, aw1) + ab1), aw2))
    t1 = jnp.sum(_dot_t(jnp.tanh(_dot_t(e1, aw1) + ab1), aw2))
    rows = lax.broadcasted_iota(jnp.int32, (8, 128), 0)
    part = jnp.where(rows == 0, t0, jnp.where(rows == 1, t1, 0.0))

    @pl.when(i == 0)
    def _():
        t_ref[...] = part

    @pl.when(i > 0)
    def _():
        t_ref[...] = t_ref[...] + part


def _han_layer(adj0, adj1, s0, s1, b0, b1, aw1, ab1, aw2):
    return pl.pallas_call(
        _han_body,
        grid=(GRID,),
        in_specs=[
            pl.BlockSpec((BLK, N), lambda i: (i, 0)),
            pl.BlockSpec((BLK, N), lambda i: (i, 0)),
            pl.BlockSpec((N, H), lambda i: (0, 0)),
            pl.BlockSpec((N, H), lambda i: (0, 0)),
            pl.BlockSpec((1, H), lambda i: (0, 0)),
            pl.BlockSpec((1, H), lambda i: (0, 0)),
            pl.BlockSpec((H, H), lambda i: (0, 0)),
            pl.BlockSpec((1, H), lambda i: (0, 0)),
            pl.BlockSpec((1, H), lambda i: (0, 0)),
        ],
        out_specs=[
            pl.BlockSpec((BLK, H), lambda i: (i, 0)),
            pl.BlockSpec((BLK, H), lambda i: (i, 0)),
            pl.BlockSpec((8, 128), lambda i: (0, 0)),
        ],
        out_shape=[
            jax.ShapeDtypeStruct((N, H), jnp.float32),
            jax.ShapeDtypeStruct((N, H), jnp.float32),
            jax.ShapeDtypeStruct((8, 128), jnp.float32),
        ],
    )(adj0, adj1, s0, s1, b0, b1, aw1, ab1, aw2)


def _combine_body(e0_ref, e1_ref, ta_ref, f0_ref, f1_ref, tb_ref,
                  h1_ref, h2_ref):
    def comb(t_ref, x0, x1):
        t = t_ref[...]
        w0 = t[0:1, 0:1] / N
        w1 = t[1:2, 0:1] / N
        m = jnp.maximum(w0, w1)
        p0 = jnp.exp(w0 - m)
        p1 = jnp.exp(w1 - m)
        s = p0 + p1
        return (p0 / s) * x0 + (p1 / s) * x1

    h1_ref[...] = comb(ta_ref, e0_ref[...], e1_ref[...])
    h2_ref[...] = comb(tb_ref, f0_ref[...], f1_ref[...])


def _combine(e0, e1, ta, f0, f1, tb):
    return pl.pallas_call(
        _combine_body,
        grid=(GRID,),
        in_specs=[
            pl.BlockSpec((BLK, H), lambda i: (i, 0)),
            pl.BlockSpec((BLK, H), lambda i: (i, 0)),
            pl.BlockSpec((8, 128), lambda i: (0, 0)),
            pl.BlockSpec((BLK, H), lambda i: (i, 0)),
            pl.BlockSpec((BLK, H), lambda i: (i, 0)),
            pl.BlockSpec((8, 128), lambda i: (0, 0)),
        ],
        out_specs=[
            pl.BlockSpec((BLK, H), lambda i: (i, 0)),
            pl.BlockSpec((BLK, H), lambda i: (i, 0)),
        ],
        out_shape=[
            jax.ShapeDtypeStruct((N, H), jnp.float32),
            jax.ShapeDtypeStruct((N, H), jnp.float32),
        ],
    )(e0, e1, ta, f0, f1, tb)


def _gather_body(d0_ref, d1_ref, h1_ref, h2_ref, f_ref):
    d0 = jnp.reshape(d0_ref[...], (BLK, 1))
    d1 = jnp.reshape(d1_ref[...], (BLK, 1))
    cols = lax.broadcasted_iota(jnp.int32, (BLK, N), 1)
    oh0 = (cols == d0).astype(jnp.float32)
    oh1 = (cols == d1).astype(jnp.float32)
    f_ref[:, :H] = _dot(oh0, h1_ref[...])
    f_ref[:, H:] = _dot(oh1, h2_ref[...])


def _pair_gather(d0, d1, h1, h2):
    return pl.pallas_call(
        _gather_body,
        grid=(GRID,),
        in_specs=[
            pl.BlockSpec((1, 1, BLK), lambda i: (i, 0, 0)),
            pl.BlockSpec((1, 1, BLK), lambda i: (i, 0, 0)),
            pl.BlockSpec((N, H), lambda i: (0, 0)),
            pl.BlockSpec((N, H), lambda i: (0, 0)),
        ],
        out_specs=pl.BlockSpec((BLK, 2 * H), lambda i: (i, 0)),
        out_shape=jax.ShapeDtypeStruct((P, 2 * H), jnp.float32),
    )(d0.reshape(GRID, 1, BLK), d1.reshape(GRID, 1, BLK), h1, h2)


def _prep_body(f_ref, w1_ref, w2_ref, w3_ref,
               fn_ref, s1_ref, s2_ref, s3_ref):
    f = f_ref[...]
    nrm = jnp.maximum(jnp.sqrt(jnp.sum(f * f, axis=1, keepdims=True)), 1e-12)
    fn = f / nrm
    fn_ref[...] = fn
    s1_ref[...] = _dot(fn, w1_ref[...])
    s2_ref[...] = _dot(fn, w2_ref[...])
    s3_ref[...] = _dot(fn, w3_ref[...])


def _prep(feature, w1, w2, w3):
    return pl.pallas_call(
        _prep_body,
        grid=(GRID,),
        in_specs=[
            pl.BlockSpec((BLK, D2), lambda i: (i, 0)),
            pl.BlockSpec((D2, D2), lambda i: (0, 0)),
            pl.BlockSpec((D2, D2), lambda i: (0, 0)),
            pl.BlockSpec((D2, D2), lambda i: (0, 0)),
        ],
        out_specs=[pl.BlockSpec((BLK, D2), lambda i: (i, 0))] * 4,
        out_shape=[jax.ShapeDtypeStruct((P, D2), jnp.float32)] * 4,
    )(feature, w1, w2, w3)


def _passa_body(fnb_ref, fn_ref, e_ref, s1_ref, s2_ref, s3_ref,
                b1_ref, b2_ref, b3_ref,
                x1_ref, x2_ref, x3_ref, idx_ref):
    sim = _dot_t(fnb_ref[...], fn_ref[...], lax.Precision.HIGHEST)
    cols = lax.broadcasted_iota(jnp.int32, (BLK, P), 1)
    kcols = lax.broadcasted_iota(jnp.int32, (BLK, 16), 1)
    onehot = jnp.zeros((BLK, P), jnp.float32)
    idxa = jnp.zeros((BLK, 16), jnp.int32)
    s = sim
    for k in range(KNN):
        m = jnp.max(s, axis=1, keepdims=True)
        first = jnp.min(jnp.where(s == m, cols, P), axis=1, keepdims=True)
        hit = cols == first
        onehot = onehot + hit.astype(jnp.float32)
        idxa = idxa + jnp.where(kcols == k, first, 0)
        s = jnp.where(hit, NEG, s)
    idx_ref[...] = idxa
    e = e_ref[...]
    x1_ref[...] = jnp.maximum(_dot(e, s1_ref[...]) + b1_ref[...], 0.0)
    x2_ref[...] = jnp.maximum(_dot(onehot, s2_ref[...]) + b2_ref[...], 0.0)
    x3_ref[...] = jnp.maximum(_dot(e * onehot, s3_ref[...]) + b3_ref[...], 0.0)


def _pass_a(fn, edge, s1, s2, s3, b1, b2, b3):
    return pl.pallas_call(
        _passa_body,
        grid=(GRID,),
        in_specs=[
            pl.BlockSpec((BLK, D2), lambda i: (i, 0)),
            pl.BlockSpec((P, D2), lambda i: (0, 0)),
            pl.BlockSpec((BLK, P), lambda i: (i, 0)),
            pl.BlockSpec((P, D2), lambda i: (0, 0)),
            pl.BlockSpec((P, D2), lambda i: (0, 0)),
            pl.BlockSpec((P, D2), lambda i: (0, 0)),
            pl.BlockSpec((1, D2), lambda i: (0, 0)),
            pl.BlockSpec((1, D2), lambda i: (0, 0)),
            pl.BlockSpec((1, D2), lambda i: (0, 0)),
        ],
        out_specs=[
            pl.BlockSpec((BLK, D2), lambda i: (i, 0)),
            pl.BlockSpec((BLK, D2), lambda i: (i, 0)),
            pl.BlockSpec((BLK, D2), lambda i: (i, 0)),
            pl.BlockSpec((BLK, 16), lambda i: (i, 0)),
        ],
        out_shape=[
            jax.ShapeDtypeStruct((P, D2), jnp.float32),
            jax.ShapeDtypeStruct((P, D2), jnp.float32),
            jax.ShapeDtypeStruct((P, D2), jnp.float32),
            jax.ShapeDtypeStruct((P, 16), jnp.int32),
        ],
    )(fn, fn, edge, s1, s2, s3, b1, b2, b3)


def _prep2_body(x1_ref, x2_ref, x3_ref, w1_ref, w2_ref, w3_ref,
                o1_ref, o2_ref, o3_ref):
    o1_ref[...] = _dot(x1_ref[...], w1_ref[...])
    o2_ref[...] = _dot(x2_ref[...], w2_ref[...])
    o3_ref[...] = _dot(x3_ref[...], w3_ref[...])


def _prep2(x1, x2, x3, w1, w2, w3):
    return pl.pallas_call(
        _prep2_body,
        grid=(GRID,),
        in_specs=[pl.BlockSpec((BLK, D2), lambda i: (i, 0))] * 3
        + [pl.BlockSpec((D2, D2), lambda i: (0, 0))] * 3,
        out_specs=[pl.BlockSpec((BLK, D2), lambda i: (i, 0))] * 3,
        out_shape=[jax.ShapeDtypeStruct((P, D2), jnp.float32)] * 3,
    )(x1, x2, x3, w1, w2, w3)


def _passb_body(e_ref, idx_ref, s1_ref, s2_ref, s3_ref,
                b1_ref, b2_ref, b3_ref, emb_ref):
    cols = lax.broadcasted_iota(jnp.int32, (BLK, P), 1)
    idxa = idx_ref[...]
    onehot = jnp.zeros((BLK, P), jnp.float32)
    for k in range(KNN):
        onehot = onehot + (cols == idxa[:, k:k + 1]).astype(jnp.float32)
    e = e_ref[...]
    emb_ref[:, :D2] = _elu(_dot(e, s1_ref[...]) + b1_ref[...])
    emb_ref[:, D2:2 * D2] = _elu(_dot(onehot, s2_ref[...]) + b2_ref[...])
    emb_ref[:, 2 * D2:] = _elu(_dot(e * onehot, s3_ref[...]) + b3_ref[...])


def _pass_b(edge, idx, s1, s2, s3, b1, b2, b3):
    return pl.pallas_call(
        _passb_body,
        grid=(GRID,),
        in_specs=[
            pl.BlockSpec((BLK, P), lambda i: (i, 0)),
            pl.BlockSpec((BLK, 16), lambda i: (i, 0)),
            pl.BlockSpec((P, D2), lambda i: (0, 0)),
            pl.BlockSpec((P, D2), lambda i: (0, 0)),
            pl.BlockSpec((P, D2), lambda i: (0, 0)),
            pl.BlockSpec((1, D2), lambda i: (0, 0)),
            pl.BlockSpec((1, D2), lambda i: (0, 0)),
            pl.BlockSpec((1, D2), lambda i: (0, 0)),
        ],
        out_specs=pl.BlockSpec((BLK, 3 * D2), lambda i: (i, 0)),
        out_shape=jax.ShapeDtypeStruct((P, 3 * D2), jnp.float32),
    )(edge, idx, s1, s2, s3, b1, b2, b3)


def _head_body(di_ref, emb_ref, w1_ref, w2_ref, out_ref):
    di = jnp.reshape(di_ref[...], (BLK, 1))
    cols = lax.broadcasted_iota(jnp.int32, (BLK, P), 1)
    oh = (cols == di).astype(jnp.float32)
    sel = _dot(oh, emb_ref[...])
    x = _elu(_dot_t(sel, w1_ref[...]))
    logits = _dot_t(x, w2_ref[...])
    m = jnp.max(logits, axis=1, keepdims=True)
    lse = m + jnp.log(jnp.sum(jnp.exp(logits - m), axis=1, keepdims=True))
    out_ref[...] = logits - lse


def _head(dsi, emb, w1, w2):
    grid = DS // BLK
    return pl.pallas_call(
        _head_body,
        grid=(grid,),
        in_specs=[
            pl.BlockSpec((1, 1, BLK), lambda i: (i, 0, 0)),
            pl.BlockSpec((P, 3 * D2), lambda i: (0, 0)),
            pl.BlockSpec((H, 3 * D2), lambda i: (0, 0)),
            pl.BlockSpec((2, H), lambda i: (0, 0)),
        ],
        out_specs=pl.BlockSpec((BLK, 2), lambda i: (i, 0)),
        out_shape=jax.ShapeDtypeStruct((DS, 2), jnp.float32),
    )(dsi.reshape(grid, 1, BLK), emb, w1, w2)


def kernel(g0_adj, g1_adj, h0, h1_feat, data, dateset_index, edge, params):
    p = params
    r = lambda b: b.reshape(1, -1)

    s00, s01 = _transform(h0, p["W_t1"],
                          p["han1_gc"][0]["W"], p["han1_gc"][1]["W"])
    s10, s11 = _transform(h1_feat, p["W_t2"],
                          p["han2_gc"][0]["W"], p["han2_gc"][1]["W"])

    a1 = p["han1_att"]
    e0, e1, ta = _han_layer(g0_adj[0], g0_adj[1], s00, s01,
                            r(p["han1_gc"][0]["b"]), r(p["han1_gc"][1]["b"]),
                            a1["W1"], r(a1["b1"]), a1["W2"])
    a2 = p["han2_att"]
    f0, f1, tb = _han_layer(g1_adj[0], g1_adj[1], s10, s11,
                            r(p["han2_gc"][0]["b"]), r(p["han2_gc"][1]["b"]),
                            a2["W1"], r(a2["b1"]), a2["W2"])

    h1, h2 = _combine(e0, e1, ta, f0, f1, tb)

    feature = _pair_gather(data[:, 0].astype(jnp.int32),
                           data[:, 1].astype(jnp.int32), h1, h2)

    g = p["clgcn"]
    fn, sp1, sp2, sp3 = _prep(feature, g[0]["gc1W"], g[1]["gc1W"],
                              g[2]["gc1W"])
    x1, x2, x3, idx = _pass_a(fn, edge, sp1, sp2, sp3,
                              r(g[0]["gc1b"]), r(g[1]["gc1b"]),
                              r(g[2]["gc1b"]))
    y1, y2, y3 = _prep2(x1, x2, x3, g[0]["gc2W"], g[1]["gc2W"], g[2]["gc2W"])
    emb = _pass_b(edge, idx, y1, y2, y3,
                  r(g[0]["gc2b"]), r(g[1]["gc2b"]), r(g[2]["gc2b"]))

    pred = _head(dateset_index.astype(jnp.int32), emb,
                 p["mlp_W1"], p["mlp_W2"])
    return (pred, h1, h2)
